# Initial kernel scaffold; baseline (speedup 1.0000x reference)
import jax
import jax.numpy as jnp
from jax.experimental import pallas as pl


def kernel(x, edge_index, batch, W1, a1s, a1d, b1, W2, a2s, a2d, b2, W3, a3s, a3d, b3, W0, b0, Wn, bn, Wl, bl):
    raise NotImplementedError("write your pallas kernel here")


# speedup vs baseline: 1.1785x; 1.1785x over previous
import functools

import jax
import jax.numpy as jnp
from jax.experimental import pallas as pl

_ET = 6600


def _dense_body(u_ref, b_ref, w_ref, a_ref, h_ref, eo_ref, *, act):
    u = u_ref[...]
    if act:
        u = jnp.maximum(u + b_ref[...], 0.0)
    h = jnp.dot(u, w_ref[...], preferred_element_type=jnp.float32)
    h_ref[...] = h
    eo_ref[...] = jnp.dot(h, a_ref[...], preferred_element_type=jnp.float32)


def _dense(u, b, W, a_s, a_d, act):
    n = u.shape[0]
    A = jnp.stack([a_s, a_d], axis=1)
    return pl.pallas_call(
        functools.partial(_dense_body, act=act),
        out_shape=(
            jax.ShapeDtypeStruct((n, W.shape[1]), jnp.float32),
            jax.ShapeDtypeStruct((n, 2), jnp.float32),
        ),
    )(u, b.reshape(1, -1), W, A)


def _logits_body(es_ref, ed_ref, e_ref):
    e = es_ref[...] + ed_ref[...]
    e_ref[...] = jnp.where(e >= 0.0, e, 0.2 * e)


def _exp_body(e_ref, m_ref, ex_ref):
    ex_ref[...] = jnp.exp(e_ref[...] - m_ref[...])


def _weight_body(ex_ref, dn_ref, hs_ref, w_ref):
    w_ref[...] = (ex_ref[...] / dn_ref[...]) * hs_ref[...]


def _edge_call(body, outs_dim, *args):
    e2 = args[0].shape[0]
    grid = e2 // _ET
    specs = [
        pl.BlockSpec((_ET, a.shape[1]), lambda i: (i, 0)) for a in args
    ]
    return pl.pallas_call(
        body,
        grid=(grid,),
        in_specs=specs,
        out_specs=pl.BlockSpec((_ET, outs_dim), lambda i: (i, 0)),
        out_shape=jax.ShapeDtypeStruct((e2, outs_dim), jnp.float32),
    )(*args)


def _bias_relu_body(u_ref, b_ref, o_ref):
    o_ref[...] = jnp.maximum(u_ref[...] + b_ref[...], 0.0)


def _head_body(hg_ref, xr_ref, w0_ref, b0_ref, wn_ref, bn_ref, wlt_ref,
               wlb_ref, bl_ref, o_ref):
    a = jnp.maximum(
        jnp.dot(hg_ref[...], w0_ref[...], preferred_element_type=jnp.float32)
        + b0_ref[...], 0.0)
    nw = jnp.maximum(
        jnp.dot(xr_ref[...], wn_ref[...], preferred_element_type=jnp.float32)
        + bn_ref[...], 0.0)
    logit = (jnp.dot(a, wlt_ref[...], preferred_element_type=jnp.float32)
             + jnp.dot(nw, wlb_ref[...], preferred_element_type=jnp.float32)
             + bl_ref[...])
    o_ref[...] = jax.nn.sigmoid(logit)


def kernel(x, edge_index, batch, W1, a1s, a1d, b1, W2, a2s, a2d, b2,
           W3, a3s, a3d, b3, W0, b0, Wn, bn, Wl, bl):
    n = x.shape[0]
    loop = jnp.arange(n, dtype=edge_index.dtype)
    src = jnp.concatenate([edge_index[0], loop])
    dst = jnp.concatenate([edge_index[1], loop])

    def gat(u, b_prev, W, a_s, a_d, act):
        h, eo = _dense(u, b_prev, W, a_s, a_d, act)
        es = eo[:, 0:1]
        ed = eo[:, 1:2]
        e = _edge_call(_logits_body, 1, es[src], ed[dst])
        m = jax.ops.segment_max(e[:, 0], dst, num_segments=n)
        ex = _edge_call(_exp_body, 1, e, m[dst][:, None])
        denom = jax.ops.segment_sum(ex[:, 0], dst, num_segments=n)
        w = _edge_call(_weight_body, h.shape[1], ex, denom[dst][:, None],
                       h[src])
        return jax.ops.segment_sum(w, dst, num_segments=n)

    zb = jnp.zeros_like(b1)
    agg1 = gat(x, zb, W1, a1s, a1d, act=False)
    agg2 = gat(agg1, b1, W2, a2s, a2d, act=True)
    agg3 = gat(agg2, b2, W3, a3s, a3d, act=True)

    h3 = pl.pallas_call(
        _bias_relu_body,
        out_shape=jax.ShapeDtypeStruct(agg3.shape, jnp.float32),
    )(agg3, b3.reshape(1, -1))

    hg = jax.ops.segment_max(h3, batch, num_segments=64)
    diffs = batch[1:] - batch[:-1]
    root = jnp.nonzero(diffs, size=63)[0]
    root = jnp.concatenate([jnp.zeros((1,), dtype=root.dtype), root + 1])
    xr = x[root]

    H = W0.shape[0]
    out = pl.pallas_call(
        _head_body,
        out_shape=jax.ShapeDtypeStruct((64, 1), jnp.float32),
    )(hg, xr, W0, b0.reshape(1, -1), Wn, bn.reshape(1, -1),
      Wl[:H], Wl[H:], bl.reshape(1, 1))
    return out
